# Initial kernel scaffold; baseline (speedup 1.0000x reference)
import jax
import jax.numpy as jnp
from jax.experimental import pallas as pl


def kernel(hidden_states, Wk, Wv, Wq, Wsf, gate_w, up_w, down_w):
    raise NotImplementedError("write your pallas kernel here")


# speedup vs baseline: 1.1073x; 1.1073x over previous
import functools

import jax
import jax.numpy as jnp
from jax.experimental import pallas as pl
from jax.experimental.pallas import tpu as pltpu

E = 8
A = 64
EA = E * A
HI = jax.lax.Precision.HIGHEST
BF = jnp.bfloat16
F32 = jnp.float32


def _r16(x):
    return x.astype(BF).astype(F32)


def _rim_body(x_ref, wsf_ref, wq_ref, wkp_ref, wv_ref, a2r_ref, margin_ref):
    xb = x_ref[...].astype(BF)
    sf = jnp.dot(xb, wsf_ref[...], preferred_element_type=F32)
    qv = jnp.dot(sf.astype(BF), wq_ref[...], preferred_element_type=F32)
    kv = jnp.dot(xb, wkp_ref[...], preferred_element_type=F32)
    vv = jnp.dot(xb, wv_ref[...], preferred_element_type=F32)

    row = jax.lax.broadcasted_iota(jnp.int32, (EA, E), 0)
    col = jax.lax.broadcasted_iota(jnp.int32, (EA, E), 1)
    segm = (row // A == col).astype(F32)

    def _roll(v, shift):
        return jnp.roll(v, shift, axis=1) if shift % v.shape[1] else v

    qvb = _r16(qv)
    kvb = _r16(kv)
    z = []
    for s in range(E):
        prod = qvb * _roll(kvb, -s * A)
        r = jnp.dot(prod, segm, precision=HI) * 0.125
        z.append(_roll(r, s))
    m = z[0]
    for s in range(1, E):
        m = jnp.maximum(m, z[s])
    es = [jnp.exp(t - m) for t in z]
    den = es[0]
    for s in range(1, E):
        den = den + es[s]
    attn = [t / den for t in es]

    vvb = _r16(vv)
    aw = jnp.zeros_like(vv)
    for t in range(E):
        g = _r16(_roll(attn[t], -t))
        b = jnp.dot(g, segm.T, precision=HI)
        aw = aw + b * _roll(vvb, -t * A)

    num = jnp.dot(jnp.exp(aw), segm, precision=HI)
    a2r_ref[...] = num / (num + float(A))
    margin_ref[...] = num - float(A)


def _mlp_body(x_ref, gw_ref, uw_ref, dw_ref, coef_ref, o_ref, acc_ref, xb_ref,
              *, nj):
    j = pl.program_id(1)

    @pl.when(j == 0)
    def _():
        xb_ref[...] = x_ref[...].astype(BF)
        acc_ref[...] = jnp.zeros_like(acc_ref)

    xb = xb_ref[...]
    g = jnp.dot(xb, gw_ref[...], preferred_element_type=F32)
    u = jnp.dot(xb, uw_ref[...], preferred_element_type=F32)
    inner = (jax.nn.silu(g) * u).astype(BF)
    acc_ref[...] += jnp.dot(inner, dw_ref[...], preferred_element_type=F32)

    @pl.when(j == nj - 1)
    def _():
        o_ref[...] = x_ref[...] + coef_ref[...] * acc_ref[...]


def _expert_mlp(hs, gw, uw, dw, coef_e, tb, fb):
    n, d = hs.shape
    ff = gw.shape[1]
    nt, nj = n // tb, ff // fb
    return pl.pallas_call(
        functools.partial(_mlp_body, nj=nj),
        grid=(nt, nj),
        in_specs=[
            pl.BlockSpec((tb, d), lambda i, j: (i, 0)),
            pl.BlockSpec((d, fb), lambda i, j: (0, j)),
            pl.BlockSpec((d, fb), lambda i, j: (0, j)),
            pl.BlockSpec((fb, d), lambda i, j: (j, 0)),
            pl.BlockSpec((tb, 1), lambda i, j: (i, 0)),
        ],
        out_specs=pl.BlockSpec((tb, d), lambda i, j: (i, 0)),
        out_shape=jax.ShapeDtypeStruct((n, d), F32),
        scratch_shapes=[pltpu.VMEM((tb, d), F32),
                        pltpu.VMEM((tb, d), BF)],
        compiler_params=pltpu.CompilerParams(
            dimension_semantics=("parallel", "arbitrary"),
            vmem_limit_bytes=100 * 1024 * 1024,
        ),
    )(hs, gw, uw, dw, coef_e)


def kernel(hidden_states, Wk, Wv, Wq, Wsf, gate_w, up_w, down_w):
    b, s, d = hidden_states.shape
    n = b * s
    hs = hidden_states.reshape(n, d)

    wkp = Wk.reshape(d, A, E).transpose(0, 2, 1).reshape(d, EA)

    tb_rim = 256
    a2r, margin = pl.pallas_call(
        _rim_body,
        grid=(n // tb_rim,),
        in_specs=[
            pl.BlockSpec((tb_rim, d), lambda i: (i, 0)),
            pl.BlockSpec((d, EA), lambda i: (0, 0)),
            pl.BlockSpec((EA, EA), lambda i: (0, 0)),
            pl.BlockSpec((d, EA), lambda i: (0, 0)),
            pl.BlockSpec((d, EA), lambda i: (0, 0)),
        ],
        out_specs=[
            pl.BlockSpec((tb_rim, E), lambda i: (i, 0)),
            pl.BlockSpec((tb_rim, E), lambda i: (i, 0)),
        ],
        out_shape=[
            jax.ShapeDtypeStruct((n, E), F32),
            jax.ShapeDtypeStruct((n, E), F32),
        ],
    )(hs, Wsf.astype(BF), Wq.astype(BF), wkp.astype(BF), Wv.astype(BF))

    mask = margin > 0.0
    coef = jnp.where(mask, a2r, 0.0)

    gate_b = gate_w.astype(BF)
    up_b = up_w.astype(BF)
    down_b = down_w.astype(BF)
    for e in range(E):
        hs = _expert_mlp(hs, gate_b[e], up_b[e], down_b[e], coef[:, e:e + 1],
                         tb=1024, fb=512)

    return hs.reshape(b, s, d), a2r, mask
